# baseline (speedup 1.0000x reference)
import dataclasses
import functools

import jax
import jax.numpy as jnp
from jax import lax
from jax.experimental import pallas as pl
from jax.experimental.pallas import tpu as pltpu
from jax.experimental.pallas import tpu_sc as plsc

_N = 10000
_E = 320000
_D = 128
_H = 8
_DH = 16
_S = 3
_SCALES = (50.0, 200.0, 500.0)
_BINS = 50

_NW = 32
_B = 128
_NCHUNK = 2560
_E_PAD = _NCHUNK * _B
_NPAD = _N + 8
_NT = 624
_NTAIL = _N - 16 * _NT


def _proj_body(x_ref, w_ref, b_ref, *o_refs):
    acc = jnp.dot(x_ref[...], w_ref[...], preferred_element_type=jnp.float32)
    acc = acc + b_ref[...]
    for i, o_ref in enumerate(o_refs):
        o_ref[...] = acc[:, i * _D:(i + 1) * _D]


def _project_qkv(x, W_all, b_all):
    BM = 1000
    P = 9 * _D
    outs = [jax.ShapeDtypeStruct((_N, _D), jnp.float32) for _ in range(9)]
    return pl.pallas_call(
        _proj_body,
        grid=(_N // BM,),
        in_specs=[
            pl.BlockSpec((BM, _D), lambda i: (i, 0)),
            pl.BlockSpec((_D, P), lambda i: (0, 0)),
            pl.BlockSpec((1, P), lambda i: (0, 0)),
        ],
        out_specs=[pl.BlockSpec((BM, _D), lambda i: (i, 0)) for _ in range(9)],
        out_shape=outs,
    )(x, W_all, b_all.reshape(1, P))


def _bias_body(ea_ref, dembT_ref, wd_ref, bd_ref, o_ref):
    ea = ea_ref[...]
    dist = ea[0:1, :]
    dx = ea[1:2, :]
    dy = ea[2:3, :]
    nrm = jnp.maximum(jnp.sqrt(dx * dx + dy * dy), 1e-8)
    ndx = dx / nrm
    ndy = dy / nrm
    BE = ea.shape[1]
    row = jax.lax.broadcasted_iota(jnp.int32, (_BINS + 1, BE), 0)
    for s in range(_S):
        bins = jnp.clip((dist * (_BINS / _SCALES[s])).astype(jnp.int32), 0, _BINS)
        oh = (row == bins).astype(jnp.float32)
        bias_d = jnp.dot(dembT_ref[s], oh, preferred_element_type=jnp.float32)
        wd = wd_ref[s]
        targ = wd[:, 0:1] * ndx + wd[:, 1:2] * ndy + bd_ref[s].reshape(_H, 1)
        o_ref[s] = bias_d + jnp.tanh(targ)


def _edge_bias(edge_attrT, dist_embT, Wd, bd):
    BE = 3200
    return pl.pallas_call(
        _bias_body,
        grid=(_E // BE,),
        in_specs=[
            pl.BlockSpec((4, BE), lambda i: (0, i)),
            pl.BlockSpec((_S, _H, _BINS + 1), lambda i: (0, 0, 0)),
            pl.BlockSpec((_S, _H, 2), lambda i: (0, 0, 0)),
            pl.BlockSpec((_S, _H), lambda i: (0, 0)),
        ],
        out_specs=pl.BlockSpec((_S, _H, BE), lambda i: (0, 0, i)),
        out_shape=jax.ShapeDtypeStruct((_S, _H, _E), jnp.float32),
    )(edge_attrT, dist_embT, Wd, bd)


def _edge_body(q0, q1, q2, k0, k1, k2, v0, v1, v2,
               cb0, cb1, cb2,
               aggr_out, denom_out,
               sd_a, sd_b, q_rows, k_a, k_b,
               ex_rows, aggr_sh, denom_sh,
               semia, semib, semq, semka, semkb):
    c = lax.axis_index("c")
    t = lax.axis_index("s")
    wid = c * 16 + t
    off = t * _NT
    lanes = lax.iota(jnp.int32, 16)
    nfull = _NCHUNK // _NW
    zero16 = jnp.zeros((16,), jnp.float32)

    for s in range(_S):
        q_hbm = (q0, q1, q2)[s]
        k_hbm = (k0, k1, k2)[s]
        v_hbm = (v0, v1, v2)[s]
        cb_hbm = (cb0, cb1, cb2)[s]

        pltpu.async_copy(cb_hbm.at[:, pl.ds(wid * _B, _B)], sd_a, semia)

        def _zq(e, carry):
            ex_rows[e, :] = zero16
            for c8 in range(8):
                q_rows[e, pl.ds(c8 * 16, 16)] = zero16
            return carry
        lax.fori_loop(0, _B, _zq, 0)

        for j in range(4):
            pltpu.sync_copy(q_rows, aggr_sh.at[pl.ds(off + j * 128, 128)])
            pltpu.sync_copy(ex_rows, denom_sh.at[pl.ds(off + j * 128, 128)])
        pltpu.sync_copy(q_rows.at[pl.ds(0, 112)],
                        aggr_sh.at[pl.ds(off + 512, 112)])
        pltpu.sync_copy(ex_rows.at[pl.ds(0, 112)],
                        denom_sh.at[pl.ds(off + 512, 112)])

        @pl.when(t == 15)
        def _zero_tail():
            pltpu.sync_copy(q_rows.at[pl.ds(0, _NTAIL)],
                            aggr_sh.at[pl.ds(16 * _NT, _NTAIL)])
            pltpu.sync_copy(ex_rows.at[pl.ds(0, _NTAIL)],
                            denom_sh.at[pl.ds(16 * _NT, _NTAIL)])

        plsc.subcore_barrier()

        def _scores_for_eighth(e, kbuf, sd):
            qrows16 = e * 16 + lanes
            for h in range(_H):
                colbase = h * 16
                score = jnp.zeros((16,), jnp.float32)
                for d in range(_DH):
                    colv = jnp.full((16,), colbase + d, jnp.int32)
                    qv = plsc.load_gather(q_rows, [qrows16, colv])
                    kv = plsc.load_gather(kbuf, [lanes, colv])
                    score = score + qv * kv
                bias16 = plsc.bitcast(sd[2 + h, pl.ds(e * 16, 16)],
                                      jnp.float32)
                ex = jnp.exp(score + bias16)
                plsc.store_scatter(
                    ex_rows, [qrows16, jnp.full((16,), h, jnp.int32)], ex)

        def process_chunk(sd, semi):
            pltpu.make_async_copy(
                cb_hbm.at[:, pl.ds(0, _B)], sd, semi).wait()
            srow = sd.at[0]
            drow = sd.at[1]
            hq = pltpu.async_copy(q_hbm.at[drow], q_rows, semq)
            pltpu.async_copy(
                k_hbm.at[srow.at[pl.ds(0, 16)]], k_a, semka)
            hq.wait()

            def sc_body(j2, carry2):
                e0 = j2 * 2
                pltpu.async_copy(
                    k_hbm.at[srow.at[pl.ds((e0 + 1) * 16, 16)]], k_b, semkb)
                pltpu.make_async_copy(
                    k_hbm.at[pl.ds(0, 16)], k_a, semka).wait()
                _scores_for_eighth(e0, k_a, sd)

                @pl.when(j2 < 3)
                def _pf_next():
                    pltpu.async_copy(
                        k_hbm.at[srow.at[pl.ds((e0 + 2) * 16, 16)]],
                        k_a, semka)

                pltpu.make_async_copy(
                    k_hbm.at[pl.ds(0, 16)], k_b, semkb).wait()
                _scores_for_eighth(e0 + 1, k_b, sd)
                return carry2

            lax.fori_loop(0, 4, sc_body, 0)

            pltpu.async_copy(v_hbm.at[srow], q_rows, semq).wait()

            def msg_body(j, carry2):
                rows16 = j * 16 + lanes

                def mh_body(h, carry3):
                    colbase = h * 16
                    ex = plsc.load_gather(
                        ex_rows, [rows16, jnp.full((16,), 0, jnp.int32) + h])
                    for d in range(_DH):
                        colv = jnp.full((16,), d, jnp.int32) + colbase
                        vv = plsc.load_gather(q_rows, [rows16, colv])
                        plsc.store_scatter(q_rows, [rows16, colv], vv * ex)
                    return carry3

                lax.fori_loop(0, _H, mh_body, 0)
                return carry2

            lax.fori_loop(0, _B // 16, msg_body, 0)
            pltpu.sync_copy(q_rows, aggr_sh.at[drow], add=True)
            pltpu.sync_copy(ex_rows, denom_sh.at[drow], add=True)

        def pair_body(p, carry):
            ch0 = 2 * p
            base1 = (wid + (ch0 + 1) * _NW) * _B
            pltpu.async_copy(cb_hbm.at[:, pl.ds(base1, _B)], sd_b, semib)
            process_chunk(sd_a, semia)

            @pl.when(ch0 + 2 < nfull)
            def _pf_a():
                base2 = (wid + (ch0 + 2) * _NW) * _B
                pltpu.async_copy(cb_hbm.at[:, pl.ds(base2, _B)], sd_a, semia)

            process_chunk(sd_b, semib)
            return carry

        lax.fori_loop(0, nfull // 2, pair_body, 0)
        plsc.subcore_barrier()

        pltpu.sync_copy(aggr_sh.at[pl.ds(off, _NT)],
                        aggr_out.at[s, c, pl.ds(off, _NT)])
        pltpu.sync_copy(denom_sh.at[pl.ds(off, _NT)],
                        denom_out.at[s, c, pl.ds(off, _NT)])

        @pl.when(t == 15)
        def _drain_tail():
            pltpu.sync_copy(aggr_sh.at[pl.ds(16 * _NT, _NTAIL)],
                            aggr_out.at[s, c, pl.ds(16 * _NT, _NTAIL)])
            pltpu.sync_copy(denom_sh.at[pl.ds(16 * _NT, _NTAIL)],
                            denom_out.at[s, c, pl.ds(16 * _NT, _NTAIL)])


def _make_edge_kernel():
    mesh = plsc.VectorSubcoreMesh(core_axis_name="c", subcore_axis_name="s")
    cp = pltpu.CompilerParams(use_tc_tiling_on_sc=False)
    if "needs_layout_passes" in pltpu.CompilerParams.__dataclass_fields__:
        cp = dataclasses.replace(cp, needs_layout_passes=False)
    return pl.kernel(
        _edge_body,
        compiler_params=cp,
        out_type=[
            jax.ShapeDtypeStruct((_S, 2, _N, _D), jnp.float32),
            jax.ShapeDtypeStruct((_S, 2, _N, 16), jnp.float32),
        ],
        mesh=mesh,
        scratch_types=[
            pltpu.VMEM((2 + _H, _B), jnp.int32),
            pltpu.VMEM((2 + _H, _B), jnp.int32),
            pltpu.VMEM((_B, _D), jnp.float32),
            pltpu.VMEM((16, _D), jnp.float32),
            pltpu.VMEM((16, _D), jnp.float32),
            pltpu.VMEM((_B, 16), jnp.float32),
            pltpu.VMEM_SHARED((_NPAD, _D), jnp.float32),
            pltpu.VMEM_SHARED((_NPAD, 16), jnp.float32),
            pltpu.SemaphoreType.DMA,
            pltpu.SemaphoreType.DMA,
            pltpu.SemaphoreType.DMA,
            pltpu.SemaphoreType.DMA,
            pltpu.SemaphoreType.DMA,
        ],
    )


def _combine_body(aggr_ref, denom_ref, wcat_ref, bcomb_ref, o_ref):
    parts = []
    for s in range(_S):
        A = aggr_ref[s, 0] + aggr_ref[s, 1]
        dn = denom_ref[s, 0, :, :_H] + denom_ref[s, 1, :, :_H]
        r = 1.0 / (dn + 1e-16)
        cols = [A[:, h * 16:(h + 1) * 16] * r[:, h:h + 1] for h in range(_H)]
        parts.append(jnp.concatenate(cols, axis=1))
    An = jnp.concatenate(parts, axis=1)
    o_ref[...] = (
        jnp.dot(An, wcat_ref[...], preferred_element_type=jnp.float32)
        + bcomb_ref[...]
    )


def _combine(aggr, denom, Wcat, bcomb):
    BM = 1000
    return pl.pallas_call(
        _combine_body,
        grid=(_N // BM,),
        in_specs=[
            pl.BlockSpec((_S, 2, BM, _D), lambda i: (0, 0, i, 0)),
            pl.BlockSpec((_S, 2, BM, 16), lambda i: (0, 0, i, 0)),
            pl.BlockSpec((_S * _D, _D), lambda i: (0, 0)),
            pl.BlockSpec((1, _D), lambda i: (0, 0)),
        ],
        out_specs=pl.BlockSpec((BM, _D), lambda i: (i, 0)),
        out_shape=jax.ShapeDtypeStruct((_N, _D), jnp.float32),
    )(aggr, denom, Wcat, bcomb.reshape(1, _D))


def kernel(x, edge_index, edge_attr, Wq, bq, Wk, bk, Wv, bv, Wo, bo, dist_emb, Wd, bd, scale_weights):
    scale = _DH ** -0.5
    W_all = jnp.concatenate(
        [jnp.concatenate([Wq[s].T * scale for s in range(_S)], axis=1),
         jnp.concatenate([Wk[s].T for s in range(_S)], axis=1),
         jnp.concatenate([Wv[s].T for s in range(_S)], axis=1)],
        axis=1,
    )
    b_all = jnp.concatenate([bq.reshape(-1) * scale, bk.reshape(-1),
                             bv.reshape(-1)], axis=0)
    tabs = _project_qkv(x, W_all, b_all)
    q_tabs, k_tabs, v_tabs = tabs[0:3], tabs[3:6], tabs[6:9]

    biasT = _edge_bias(edge_attr.T, jnp.transpose(dist_emb, (0, 2, 1)), Wd, bd)

    bias_bits = jax.lax.bitcast_convert_type(biasT, jnp.int32)
    pad_sd = jnp.concatenate(
        [jnp.zeros((1, _E_PAD - _E), jnp.int32),
         jnp.full((1, _E_PAD - _E), _N, jnp.int32),
         jnp.zeros((_H, _E_PAD - _E), jnp.int32)], axis=0)
    cbs = [jnp.concatenate(
        [jnp.concatenate([edge_index, bias_bits[s]], axis=0), pad_sd], axis=1)
        for s in range(_S)]
    q_tabs = [jnp.pad(q, ((0, _NPAD - _N), (0, 0))) for q in q_tabs]
    k_tabs = [jnp.pad(k, ((0, _NPAD - _N), (0, 0))) for k in k_tabs]
    v_tabs = [jnp.pad(v, ((0, _NPAD - _N), (0, 0))) for v in v_tabs]

    edge_kernel = _make_edge_kernel()
    aggr, denom = edge_kernel(
        q_tabs[0], q_tabs[1], q_tabs[2],
        k_tabs[0], k_tabs[1], k_tabs[2],
        v_tabs[0], v_tabs[1], v_tabs[2],
        cbs[0], cbs[1], cbs[2])

    w = jax.nn.softmax(scale_weights)
    Wcat = jnp.concatenate([w[s] * Wo[s].T for s in range(_S)], axis=0)
    bcomb = (w[:, None] * bo).sum(axis=0)
    return _combine(aggr, denom, Wcat, bcomb)


# speedup vs baseline: 1.0427x; 1.0427x over previous
import dataclasses
import functools

import jax
import jax.numpy as jnp
from jax import lax
from jax.experimental import pallas as pl
from jax.experimental.pallas import tpu as pltpu
from jax.experimental.pallas import tpu_sc as plsc

_N = 10000
_E = 320000
_D = 128
_H = 8
_DH = 16
_S = 3
_SCALES = (50.0, 200.0, 500.0)
_BINS = 50

_NW = 32
_B = 128
_NCHUNK_TOT = _E // _B
_NT = 624
_NTAIL = _N - 16 * _NT


def _proj_body(x_ref, w_ref, b_ref, *o_refs):
    acc = jnp.dot(x_ref[...], w_ref[...], preferred_element_type=jnp.float32)
    acc = acc + b_ref[...]
    for i, o_ref in enumerate(o_refs):
        o_ref[...] = acc[:, i * _D:(i + 1) * _D]


def _project_qkv(x, W_all, b_all):
    BM = 1000
    P = 9 * _D
    outs = [jax.ShapeDtypeStruct((_N, _D), jnp.float32) for _ in range(9)]
    return pl.pallas_call(
        _proj_body,
        grid=(_N // BM,),
        in_specs=[
            pl.BlockSpec((BM, _D), lambda i: (i, 0)),
            pl.BlockSpec((_D, P), lambda i: (0, 0)),
            pl.BlockSpec((1, P), lambda i: (0, 0)),
        ],
        out_specs=[pl.BlockSpec((BM, _D), lambda i: (i, 0)) for _ in range(9)],
        out_shape=outs,
    )(x, W_all, b_all.reshape(1, P))


def _bias_body(ea_ref, dembT_ref, wd_ref, bd_ref, o_ref):
    ea = ea_ref[...]
    dist = ea[0:1, :]
    dx = ea[1:2, :]
    dy = ea[2:3, :]
    nrm = jnp.maximum(jnp.sqrt(dx * dx + dy * dy), 1e-8)
    ndx = dx / nrm
    ndy = dy / nrm
    BE = ea.shape[1]
    row = jax.lax.broadcasted_iota(jnp.int32, (_BINS + 1, BE), 0)
    for s in range(_S):
        bins = jnp.clip((dist * (_BINS / _SCALES[s])).astype(jnp.int32), 0, _BINS)
        oh = (row == bins).astype(jnp.float32)
        bias_d = jnp.dot(dembT_ref[s], oh, preferred_element_type=jnp.float32)
        wd = wd_ref[s]
        targ = wd[:, 0:1] * ndx + wd[:, 1:2] * ndy + bd_ref[s].reshape(_H, 1)
        o_ref[s] = bias_d + jnp.tanh(targ)


def _edge_bias(edge_attrT, dist_embT, Wd, bd):
    BE = 3200
    return pl.pallas_call(
        _bias_body,
        grid=(_E // BE,),
        in_specs=[
            pl.BlockSpec((4, BE), lambda i: (0, i)),
            pl.BlockSpec((_S, _H, _BINS + 1), lambda i: (0, 0, 0)),
            pl.BlockSpec((_S, _H, 2), lambda i: (0, 0, 0)),
            pl.BlockSpec((_S, _H), lambda i: (0, 0)),
        ],
        out_specs=pl.BlockSpec((_S, _H, BE), lambda i: (0, 0, i)),
        out_shape=jax.ShapeDtypeStruct((_S, _H, _E), jnp.float32),
    )(edge_attrT, dist_embT, Wd, bd)


def _edge_body(q0, q1, q2, k0, k1, k2, v0, v1, v2,
               cb0, cb1, cb2,
               aggr_out, denom_out,
               sd_v, q_rows, k_q,
               ex_rows, aggr_sh, denom_sh, sem0, sem1):
    c = lax.axis_index("c")
    t = lax.axis_index("s")
    wid = c * 16 + t
    off = t * _NT
    lanes = lax.iota(jnp.int32, 16)
    nfull = _NCHUNK_TOT // _NW
    nchunks = nfull + jnp.where(wid < _NCHUNK_TOT - nfull * _NW, 1, 0)
    zero16 = jnp.zeros((16,), jnp.float32)

    for s in range(_S):
        q_hbm = (q0, q1, q2)[s]
        k_hbm = (k0, k1, k2)[s]
        v_hbm = (v0, v1, v2)[s]
        cb_hbm = (cb0, cb1, cb2)[s]

        def _zq(e, carry):
            ex_rows[e, :] = zero16
            for c8 in range(8):
                q_rows[e, pl.ds(c8 * 16, 16)] = zero16
            return carry
        lax.fori_loop(0, _B, _zq, 0)

        for j in range(4):
            pltpu.sync_copy(q_rows, aggr_sh.at[pl.ds(off + j * 128, 128)])
            pltpu.sync_copy(ex_rows, denom_sh.at[pl.ds(off + j * 128, 128)])
        pltpu.sync_copy(q_rows.at[pl.ds(0, 112)],
                        aggr_sh.at[pl.ds(off + 512, 112)])
        pltpu.sync_copy(ex_rows.at[pl.ds(0, 112)],
                        denom_sh.at[pl.ds(off + 512, 112)])

        @pl.when(t == 15)
        def _zero_tail():
            pltpu.sync_copy(q_rows.at[pl.ds(0, _NTAIL)],
                            aggr_sh.at[pl.ds(16 * _NT, _NTAIL)])
            pltpu.sync_copy(ex_rows.at[pl.ds(0, _NTAIL)],
                            denom_sh.at[pl.ds(16 * _NT, _NTAIL)])

        plsc.subcore_barrier()

        def chunk_body(k, carry):
            base = (wid + k * _NW) * _B
            pltpu.sync_copy(cb_hbm.at[:, pl.ds(base, _B)], sd_v)
            srow = sd_v.at[0]
            drow = sd_v.at[1]

            pltpu.async_copy(q_hbm.at[drow], q_rows, sem0).wait()

            for quarter in range(4):
                qb = quarter * 32
                pltpu.async_copy(
                    k_hbm.at[srow.at[pl.ds(qb, 32)]], k_q, sem1).wait()

                def score_body(j, carry2):
                    rows16 = j * 16 + lanes
                    qrows16 = qb + rows16
                    for h in range(_H):
                        colbase = h * 16
                        score = jnp.zeros((16,), jnp.float32)
                        for d in range(_DH):
                            colv = jnp.full((16,), colbase + d, jnp.int32)
                            qv = plsc.load_gather(q_rows, [qrows16, colv])
                            kv = plsc.load_gather(k_q, [rows16, colv])
                            score = score + qv * kv
                        bias16 = plsc.bitcast(
                            sd_v[2 + h, pl.ds(qb + j * 16, 16)], jnp.float32)
                        ex = jnp.exp(score + bias16)
                        plsc.store_scatter(
                            ex_rows,
                            [qrows16, jnp.full((16,), h, jnp.int32)], ex)
                    return carry2

                lax.fori_loop(0, 2, score_body, 0)

            pltpu.async_copy(v_hbm.at[srow], q_rows, sem0).wait()

            def msg_body(j, carry2):
                rows16 = j * 16 + lanes
                for h in range(_H):
                    colbase = h * 16
                    ex = plsc.load_gather(
                        ex_rows, [rows16, jnp.full((16,), h, jnp.int32)])
                    for d in range(_DH):
                        colv = jnp.full((16,), colbase + d, jnp.int32)
                        vv = plsc.load_gather(q_rows, [rows16, colv])
                        plsc.store_scatter(q_rows, [rows16, colv], vv * ex)
                return carry2

            lax.fori_loop(0, _B // 16, msg_body, 0)
            pltpu.sync_copy(q_rows, aggr_sh.at[drow], add=True)
            pltpu.sync_copy(ex_rows, denom_sh.at[drow], add=True)
            return carry

        lax.fori_loop(0, nchunks, chunk_body, 0)
        plsc.subcore_barrier()

        pltpu.sync_copy(aggr_sh.at[pl.ds(off, _NT)],
                        aggr_out.at[s, c, pl.ds(off, _NT)])
        pltpu.sync_copy(denom_sh.at[pl.ds(off, _NT)],
                        denom_out.at[s, c, pl.ds(off, _NT)])

        @pl.when(t == 15)
        def _drain_tail():
            pltpu.sync_copy(aggr_sh.at[pl.ds(16 * _NT, _NTAIL)],
                            aggr_out.at[s, c, pl.ds(16 * _NT, _NTAIL)])
            pltpu.sync_copy(denom_sh.at[pl.ds(16 * _NT, _NTAIL)],
                            denom_out.at[s, c, pl.ds(16 * _NT, _NTAIL)])


def _make_edge_kernel():
    mesh = plsc.VectorSubcoreMesh(core_axis_name="c", subcore_axis_name="s")
    cp = pltpu.CompilerParams(use_tc_tiling_on_sc=False)
    if "needs_layout_passes" in pltpu.CompilerParams.__dataclass_fields__:
        cp = dataclasses.replace(cp, needs_layout_passes=False)
    return pl.kernel(
        _edge_body,
        compiler_params=cp,
        out_type=[
            jax.ShapeDtypeStruct((_S, 2, _N, _D), jnp.float32),
            jax.ShapeDtypeStruct((_S, 2, _N, 16), jnp.float32),
        ],
        mesh=mesh,
        scratch_types=[
            pltpu.VMEM((2 + _H, _B), jnp.int32),
            pltpu.VMEM((_B, _D), jnp.float32),
            pltpu.VMEM((32, _D), jnp.float32),
            pltpu.VMEM((_B, 16), jnp.float32),
            pltpu.VMEM_SHARED((_N, _D), jnp.float32),
            pltpu.VMEM_SHARED((_N, 16), jnp.float32),
            pltpu.SemaphoreType.DMA,
            pltpu.SemaphoreType.DMA,
        ],
    )


def _combine_body(aggr_ref, denom_ref, wcat_ref, bcomb_ref, o_ref):
    parts = []
    for s in range(_S):
        A = aggr_ref[s, 0] + aggr_ref[s, 1]
        dn = denom_ref[s, 0, :, :_H] + denom_ref[s, 1, :, :_H]
        r = 1.0 / (dn + 1e-16)
        cols = [A[:, h * 16:(h + 1) * 16] * r[:, h:h + 1] for h in range(_H)]
        parts.append(jnp.concatenate(cols, axis=1))
    An = jnp.concatenate(parts, axis=1)
    o_ref[...] = (
        jnp.dot(An, wcat_ref[...], preferred_element_type=jnp.float32)
        + bcomb_ref[...]
    )


def _combine(aggr, denom, Wcat, bcomb):
    BM = 1000
    return pl.pallas_call(
        _combine_body,
        grid=(_N // BM,),
        in_specs=[
            pl.BlockSpec((_S, 2, BM, _D), lambda i: (0, 0, i, 0)),
            pl.BlockSpec((_S, 2, BM, 16), lambda i: (0, 0, i, 0)),
            pl.BlockSpec((_S * _D, _D), lambda i: (0, 0)),
            pl.BlockSpec((1, _D), lambda i: (0, 0)),
        ],
        out_specs=pl.BlockSpec((BM, _D), lambda i: (i, 0)),
        out_shape=jax.ShapeDtypeStruct((_N, _D), jnp.float32),
    )(aggr, denom, Wcat, bcomb.reshape(1, _D))


def kernel(x, edge_index, edge_attr, Wq, bq, Wk, bk, Wv, bv, Wo, bo, dist_emb, Wd, bd, scale_weights):
    scale = _DH ** -0.5
    W_all = jnp.concatenate(
        [jnp.concatenate([Wq[s].T * scale for s in range(_S)], axis=1),
         jnp.concatenate([Wk[s].T for s in range(_S)], axis=1),
         jnp.concatenate([Wv[s].T for s in range(_S)], axis=1)],
        axis=1,
    )
    b_all = jnp.concatenate([bq.reshape(-1) * scale, bk.reshape(-1),
                             bv.reshape(-1)], axis=0)
    tabs = _project_qkv(x, W_all, b_all)
    q_tabs, k_tabs, v_tabs = tabs[0:3], tabs[3:6], tabs[6:9]

    biasT = _edge_bias(edge_attr.T, jnp.transpose(dist_emb, (0, 2, 1)), Wd, bd)

    bias_bits = jax.lax.bitcast_convert_type(biasT, jnp.int32)
    cbs = [jnp.concatenate([edge_index, bias_bits[s]], axis=0)
           for s in range(_S)]

    edge_kernel = _make_edge_kernel()
    aggr, denom = edge_kernel(
        q_tabs[0], q_tabs[1], q_tabs[2],
        k_tabs[0], k_tabs[1], k_tabs[2],
        v_tabs[0], v_tabs[1], v_tabs[2],
        cbs[0], cbs[1], cbs[2])

    w = jax.nn.softmax(scale_weights)
    Wcat = jnp.concatenate([w[s] * Wo[s].T for s in range(_S)], axis=0)
    bcomb = (w[:, None] * bo).sum(axis=0)
    return _combine(aggr, denom, Wcat, bcomb)
